# baseline (speedup 1.0000x reference)
import functools

import jax
import jax.numpy as jnp
from jax import lax
from jax.experimental import pallas as pl
from jax.experimental.pallas import tpu as pltpu
from jax.experimental.pallas import tpu_sc as plsc

_NC = 2
_NS = 16
_B = 16
_Q = 300
_QP = 384
_C = 92
_L = 16
_G = 10


def _combine(ma, la, mb, lb):
    m = jnp.maximum(ma, mb)
    l = jnp.where(mb > ma, lb, la)
    return m, jnp.where(mb == ma, jnp.minimum(la, lb), l)


def _sc_body(lt_hbm, sclb_hbm, lslab, sslab, llab):
    wid = lax.axis_index("s") * _NC + lax.axis_index("c")
    b = wid // 2
    half = wid % 2
    @pl.when(half == 0)
    def _():
        pltpu.sync_copy(lt_hbm.at[:, b, pl.ds(0, _G * _L)], lslab)

    @pl.when(half == 1)
    def _():
        pltpu.sync_copy(lt_hbm.at[:, b, pl.ds(144, _G * _L)], lslab)

    zeros = jnp.zeros((_L,), jnp.float32)
    neg = jnp.full((_L,), -jnp.inf, jnp.float32)

    def group(g, carry):
        qo = pl.multiple_of(g * _L, _L)

        def p1(i, acc):
            m0, m1, m2, m3, l0, l1, l2, l3, cf = acc
            c = i * 4
            v0 = lslab[c, pl.ds(qo, _L)]
            v1 = lslab[c + 1, pl.ds(qo, _L)]
            v2 = lslab[c + 2, pl.ds(qo, _L)]
            v3 = lslab[c + 3, pl.ds(qo, _L)]
            l0 = jnp.where(v0 > m0, cf, l0)
            l1 = jnp.where(v1 > m1, cf + 1.0, l1)
            l2 = jnp.where(v2 > m2, cf + 2.0, l2)
            l3 = jnp.where(v3 > m3, cf + 3.0, l3)
            return (
                jnp.maximum(m0, v0), jnp.maximum(m1, v1),
                jnp.maximum(m2, v2), jnp.maximum(m3, v3),
                l0, l1, l2, l3, cf + 4.0,
            )

        init = (neg, neg, neg, neg, zeros, zeros, zeros, zeros, zeros)
        m0, m1, m2, m3, l0, l1, l2, l3, cf = lax.fori_loop(0, 22, p1, init)
        v0 = lslab[88, pl.ds(qo, _L)]
        v1 = lslab[89, pl.ds(qo, _L)]
        v2 = lslab[90, pl.ds(qo, _L)]
        l0 = jnp.where(v0 > m0, cf, l0)
        l1 = jnp.where(v1 > m1, cf + 1.0, l1)
        l2 = jnp.where(v2 > m2, cf + 2.0, l2)
        m0 = jnp.maximum(m0, v0)
        m1 = jnp.maximum(m1, v1)
        m2 = jnp.maximum(m2, v2)
        ma, la = _combine(m0, l0, m1, l1)
        mb, lb = _combine(m2, l2, m3, l3)
        m91, lbl = _combine(ma, la, mb, lb)
        mall = jnp.maximum(m91, lslab[91, pl.ds(qo, _L)])

        def p2(i, acc):
            s0, s1, s2, s3 = acc
            c = i * 4
            return (
                s0 + jnp.exp(lslab[c, pl.ds(qo, _L)] - mall),
                s1 + jnp.exp(lslab[c + 1, pl.ds(qo, _L)] - mall),
                s2 + jnp.exp(lslab[c + 2, pl.ds(qo, _L)] - mall),
                s3 + jnp.exp(lslab[c + 3, pl.ds(qo, _L)] - mall),
            )

        s0, s1, s2, s3 = lax.fori_loop(0, 23, p2, (zeros, zeros, zeros, zeros))
        s = (s0 + s1) + (s2 + s3)
        sslab[pl.ds(qo, _L)] = jnp.exp(m91 - mall) / s
        llab[pl.ds(qo, _L)] = lbl
        return carry

    lax.fori_loop(0, _G, group, 0)

    @pl.when(half == 0)
    def _():
        pltpu.sync_copy(sslab, sclb_hbm.at[0, b, pl.ds(0, _G * _L)])
        pltpu.sync_copy(llab, sclb_hbm.at[1, b, pl.ds(0, _G * _L)])

    @pl.when(half == 1)
    def _():
        pltpu.sync_copy(sslab, sclb_hbm.at[0, b, pl.ds(144, _G * _L)])
        pltpu.sync_copy(llab, sclb_hbm.at[1, b, pl.ds(144, _G * _L)])


_sc_softmax = functools.partial(
    pl.kernel,
    mesh=plsc.VectorSubcoreMesh(core_axis_name="c", subcore_axis_name="s"),
    out_type=jax.ShapeDtypeStruct((2, _B, _QP), jnp.float32),
    compiler_params=pltpu.CompilerParams(
        use_tc_tiling_on_sc=False, needs_layout_passes=False
    ),
    scratch_types=[
        pltpu.VMEM((_C, _G * _L), jnp.float32),
        pltpu.VMEM((_G * _L,), jnp.float32),
        pltpu.VMEM((_G * _L,), jnp.float32),
    ],
)(_sc_body)


def _tc_body(bt_ref, sclb_ref, tsf_ref, conf_ref, *out_refs):
    bt = bt_ref[...]
    tsf = tsf_ref[...]
    cx = bt[:, 0, :]
    cy = bt[:, 1, :]
    w = bt[:, 2, :]
    h = bt[:, 3, :]
    img_h = tsf[:, 0:1]
    img_w = tsf[:, 1:2]
    y1 = (cy - 0.5 * h) * img_h
    x1 = (cx - 0.5 * w) * img_w
    y2 = (cy + 0.5 * h) * img_h
    x2 = (cx + 0.5 * w) * img_w
    sclb = sclb_ref[...]
    conf = conf_ref[0, 0]
    for i in range(_B):
        sc = jnp.reshape(sclb[3 * i:3 * i + 3, :], (_QP,))[:_Q]
        lb = jnp.reshape(sclb[48 + 3 * i:48 + 3 * i + 3, :], (_QP,))[:_Q]
        keep = sc > conf
        leaf = jnp.stack([y1[i], x1[i], y2[i], x2[i], sc, lb], axis=0)
        out_refs[i][...] = jnp.where(keep[None, :], leaf, 0.0)


def kernel(pred_logits, pred_boxes, target_sizes, confidence):
    lt = jnp.pad(pred_logits, ((0, 0), (0, _QP - _Q), (0, 0)))
    lt = jnp.transpose(lt, (2, 0, 1))
    sclb = _sc_softmax(lt).reshape(6 * _B, 128)
    bt = jnp.transpose(pred_boxes, (0, 2, 1))
    tsf = target_sizes.astype(jnp.float32)
    conf = jnp.asarray(confidence, jnp.float32).reshape(1, 1)
    outs = pl.pallas_call(
        _tc_body,
        out_shape=tuple(
            jax.ShapeDtypeStruct((6, _Q), jnp.float32) for _ in range(_B)
        ),
    )(bt, sclb, tsf, conf)
    return tuple(jnp.transpose(o) for o in outs)
